# Initial kernel scaffold; baseline (speedup 1.0000x reference)
import jax
import jax.numpy as jnp
from jax.experimental import pallas as pl


def kernel(x, Wq, Wk):
    raise NotImplementedError("write your pallas kernel here")


# speedup vs baseline: 17.4560x; 17.4560x over previous
import jax
import jax.numpy as jnp
from jax import lax
from jax.experimental import pallas as pl
from jax.experimental.pallas import tpu as pltpu

_WINDOW = 128
_QBLK = 128
_NEG_INF = float("-inf")
_ROW_CHUNK = 8

_DN = (((1,), (1,)), ((), ()))


def _mask_body(xq_ref, xlo_ref, wq_ref, wk_ref, out_ref):
    i = pl.program_id(1)
    xq = xq_ref[0]
    xlo = xlo_ref[0]
    wq = wq_ref[...]
    wk = wk_ref[...]

    q = lax.dot_general(xq, wq, _DN, preferred_element_type=jnp.float32)
    k_hi = lax.dot_general(xq, wk, _DN, preferred_element_type=jnp.float32)
    k_lo = lax.dot_general(xlo, wk, _DN, preferred_element_type=jnp.float32)
    s_lo = lax.dot_general(q, k_lo, _DN, preferred_element_type=jnp.float32)
    s_hi = lax.dot_general(q, k_hi, _DN, preferred_element_type=jnp.float32)
    v = jnp.maximum(jnp.concatenate([s_lo, s_hi], axis=1), 0.0)

    r = lax.broadcasted_iota(jnp.int32, (_QBLK, 2 * _QBLK), 0)
    c = lax.broadcasted_iota(jnp.int32, (_QBLK, 2 * _QBLK), 1)
    valid = (c >= r + 1) & (c <= r + _WINDOW) & (_QBLK * (i - 1) + c >= 0)
    vm = jnp.where(valid, v, -1.0)

    rr = lax.broadcasted_iota(jnp.int32, (_QBLK, 1), 0)
    length = jnp.minimum(_QBLK * i + rr + 1, _WINDOW)
    kk = jnp.maximum(1, length // 2)

    ci = lax.broadcasted_iota(jnp.int32, (_ROW_CHUNK, 2 * _QBLK, 2 * _QBLK), 2)
    cj = lax.broadcasted_iota(jnp.int32, (_ROW_CHUNK, 2 * _QBLK, 2 * _QBLK), 1)
    i_lt_j = ci < cj

    out_ref[...] = jnp.full(out_ref.shape, _NEG_INF, jnp.float32)
    start_lo = _QBLK * lax.max(i - 1, 0)

    for c0 in range(0, _QBLK, _ROW_CHUNK):
        a = vm[c0:c0 + _ROW_CHUNK]
        aj = a[:, :, None]
        ai = a[:, None, :]
        beats = (ai > aj) | ((ai == aj) & i_lt_j)
        rank = jnp.sum(beats.astype(jnp.int32), axis=2)
        sel = valid[c0:c0 + _ROW_CHUNK] & (rank < kk[c0:c0 + _ROW_CHUNK])
        vals = jnp.where(sel, 0.0, _NEG_INF).astype(jnp.float32)
        rows = pl.ds(c0, _ROW_CHUNK)
        out_ref[0, 0, rows, pl.ds(start_lo, _QBLK)] = vals[:, :_QBLK]
        out_ref[0, 0, rows, pl.ds(_QBLK * i, _QBLK)] = vals[:, _QBLK:]


def kernel(x, Wq, Wk):
    B, S, D = x.shape
    idim = Wq.shape[0]
    nblk = S // _QBLK
    grid = (B, nblk)
    out = pl.pallas_call(
        _mask_body,
        grid=grid,
        in_specs=[
            pl.BlockSpec((1, _QBLK, D), lambda b, i: (b, i, 0)),
            pl.BlockSpec((1, _QBLK, D), lambda b, i: (b, lax.max(i - 1, 0), 0)),
            pl.BlockSpec((idim, D), lambda b, i: (0, 0)),
            pl.BlockSpec((idim, D), lambda b, i: (0, 0)),
        ],
        out_specs=pl.BlockSpec((1, 1, _QBLK, S), lambda b, i: (b, 0, i, 0)),
        out_shape=jax.ShapeDtypeStruct((B, 1, S, S), jnp.float32),
        compiler_params=pltpu.CompilerParams(
            dimension_semantics=("parallel", "arbitrary"),
        ),
    )(x, x, Wq, Wk)
    return out


# speedup vs baseline: 27.1336x; 1.5544x over previous
import jax
import jax.numpy as jnp
from jax import lax
from jax.experimental import pallas as pl
from jax.experimental.pallas import tpu as pltpu
from jax.experimental.pallas import tpu_sc as plsc

_WINDOW = 128
_QBLK = 128
_NEG_INF = float("-inf")
_ROW_CHUNK = 8

_DN = (((1,), (1,)), ((), ()))


def _row_roll(x, left):
    rr = lax.broadcasted_iota(jnp.int32, (_QBLK, 1), 0)

    def roll(arr, amt):
        if left:
            return jnp.concatenate([arr[:, amt:], arr[:, :amt]], axis=1)
        return jnp.concatenate([arr[:, -amt:], arr[:, :-amt]], axis=1)

    for b in range(7):
        pred = ((rr >> b) & 1) == 1
        x = jnp.where(pred, roll(x, 1 << b), x)
    return roll(x, 1)


def _band_body(xq_ref, xlo_ref, wq_ref, wk_ref, band_ref):
    i = pl.program_id(1)
    xq = xq_ref[0]
    xlo = xlo_ref[0]
    wq = wq_ref[...]
    wk = wk_ref[...]

    q = lax.dot_general(xq, wq, _DN, preferred_element_type=jnp.float32)
    k_hi = lax.dot_general(xq, wk, _DN, preferred_element_type=jnp.float32)
    k_lo = lax.dot_general(xlo, wk, _DN, preferred_element_type=jnp.float32)
    s_lo = lax.dot_general(q, k_lo, _DN, preferred_element_type=jnp.float32)
    s_hi = lax.dot_general(q, k_hi, _DN, preferred_element_type=jnp.float32)
    v = jnp.maximum(jnp.concatenate([s_lo, s_hi], axis=1), 0.0)

    r = lax.broadcasted_iota(jnp.int32, (_QBLK, 2 * _QBLK), 0)
    c = lax.broadcasted_iota(jnp.int32, (_QBLK, 2 * _QBLK), 1)
    valid = (c >= r + 1) & (c <= r + _WINDOW) & (_QBLK * (i - 1) + c >= 0)
    vm = jnp.where(valid, v, -1.0)

    w = _row_roll(vm, left=True)[:, :_QBLK]

    rr = lax.broadcasted_iota(jnp.int32, (_QBLK, 1), 0)
    length = jnp.minimum(_QBLK * i + rr + 1, _WINDOW)
    kk = jnp.maximum(1, length // 2)

    ci = lax.broadcasted_iota(jnp.int32, (_ROW_CHUNK, _QBLK, _QBLK), 2)
    cj = lax.broadcasted_iota(jnp.int32, (_ROW_CHUNK, _QBLK, _QBLK), 1)
    i_lt_j = ci < cj

    chunks = []
    for c0 in range(0, _QBLK, _ROW_CHUNK):
        a = w[c0:c0 + _ROW_CHUNK]
        aj = a[:, :, None]
        ai = a[:, None, :]
        beats = (ai > aj) | ((ai == aj) & i_lt_j)
        rank = jnp.sum(beats.astype(jnp.int32), axis=2)
        s_chunk = (a >= 0.0) & (rank < kk[c0:c0 + _ROW_CHUNK])
        chunks.append(s_chunk.astype(jnp.float32))
    sel = jnp.concatenate(chunks, axis=0)

    padded = jnp.concatenate([sel, jnp.zeros((_QBLK, _QBLK), jnp.float32)],
                             axis=1)
    sel_band = _row_roll(padded, left=False)
    vals = jnp.where(sel_band > 0.5, 0.0, _NEG_INF).astype(jnp.float32)

    shifted = jnp.concatenate(
        [vals[:, _QBLK:], jnp.full((_QBLK, _QBLK), _NEG_INF, jnp.float32)],
        axis=1)
    band_ref[...] = jnp.where(i == 0, shifted, vals)


def _sc_body(band_hbm, out_hbm, band_v, tmpl, lsem, tsem, bsem):
    wid = lax.axis_index("s") * 2 + lax.axis_index("c")
    gbase = wid * _QBLK
    i_blk = lax.rem(wid, 16)
    coff = _QBLK * lax.max(i_blk - 1, 0)

    load = pltpu.async_copy(band_hbm.at[pl.ds(gbase, _QBLK)], band_v, lsem)

    neg = jnp.full((16,), _NEG_INF, jnp.float32)

    def fill(j, _):
        for rb in range(8):
            tmpl[rb, pl.ds(j * 16, 16)] = neg
        return _

    lax.fori_loop(0, 128, fill, None)

    blankets = [
        pltpu.async_copy(tmpl, out_hbm.at[pl.ds(gbase + 8 * t, 8)], tsem)
        for t in range(16)
    ]
    load.wait()
    for cp in blankets:
        cp.wait()
    pltpu.async_copy(
        band_v, out_hbm.at[pl.ds(gbase, _QBLK), pl.ds(coff, 2 * _QBLK)],
        bsem).wait()


def kernel(x, Wq, Wk):
    B, S, D = x.shape
    idim = Wq.shape[0]
    nblk = S // _QBLK
    band = pl.pallas_call(
        _band_body,
        grid=(B, nblk),
        in_specs=[
            pl.BlockSpec((1, _QBLK, D), lambda b, i: (b, i, 0)),
            pl.BlockSpec((1, _QBLK, D), lambda b, i: (b, lax.max(i - 1, 0), 0)),
            pl.BlockSpec((idim, D), lambda b, i: (0, 0)),
            pl.BlockSpec((idim, D), lambda b, i: (0, 0)),
        ],
        out_specs=pl.BlockSpec((_QBLK, 2 * _QBLK), lambda b, i: (b * nblk + i, 0)),
        out_shape=jax.ShapeDtypeStruct((B * S, 2 * _QBLK), jnp.float32),
        compiler_params=pltpu.CompilerParams(
            dimension_semantics=("parallel", "arbitrary"),
        ),
    )(x, x, Wq, Wk)

    sc_construct = pl.kernel(
        _sc_body,
        out_type=jax.ShapeDtypeStruct((B * S, S), jnp.float32),
        mesh=plsc.VectorSubcoreMesh(core_axis_name="c", subcore_axis_name="s",
                                    num_cores=2),
        scratch_types=[
            pltpu.VMEM((_QBLK, 2 * _QBLK), jnp.float32),
            pltpu.VMEM((8, S), jnp.float32),
            pltpu.SemaphoreType.DMA,
            pltpu.SemaphoreType.DMA,
            pltpu.SemaphoreType.DMA,
        ],
    )
    mask2d = sc_construct(band)
    return mask2d.reshape(B, 1, S, S)


# speedup vs baseline: 59.7700x; 2.2028x over previous
import jax
import jax.numpy as jnp
from jax import lax
from jax.experimental import pallas as pl
from jax.experimental.pallas import tpu as pltpu
from jax.experimental.pallas import tpu_sc as plsc

_WINDOW = 128
_QBLK = 128
_NEG_INF = float("-inf")
_ROW_CHUNK = 8

_DN = (((1,), (1,)), ((), ()))


def _band_body(xq_ref, xlo_ref, wq_ref, wk_ref, band_ref):
    i = pl.program_id(1)
    xq = xq_ref[0]
    xlo = xlo_ref[0]
    wq = wq_ref[...]
    wk = wk_ref[...]

    q = lax.dot_general(xq, wq, _DN, preferred_element_type=jnp.float32)
    k_hi = lax.dot_general(xq, wk, _DN, preferred_element_type=jnp.float32)
    k_lo = lax.dot_general(xlo, wk, _DN, preferred_element_type=jnp.float32)
    s_lo = lax.dot_general(q, k_lo, _DN, preferred_element_type=jnp.float32)
    s_hi = lax.dot_general(q, k_hi, _DN, preferred_element_type=jnp.float32)
    v = jnp.maximum(jnp.concatenate([s_lo, s_hi], axis=1), 0.0)

    r = lax.broadcasted_iota(jnp.int32, (_QBLK, 2 * _QBLK), 0)
    c = lax.broadcasted_iota(jnp.int32, (_QBLK, 2 * _QBLK), 1)
    valid = (c >= r + 1) & (c <= r + _WINDOW) & (_QBLK * (i - 1) + c >= 0)
    vm = jnp.where(valid, v, -1.0)

    rr = lax.broadcasted_iota(jnp.int32, (_QBLK, 1), 0)
    length = jnp.minimum(_QBLK * i + rr + 1, _WINDOW)
    kk = jnp.maximum(1, length // 2)
    kkf = kk.astype(jnp.float32)

    u = lax.bitcast_convert_type(vm + 0.0, jnp.int32)

    def step(_, lohi):
        lo, hi = lohi
        mid = lo + lax.shift_right_logical(hi - lo + 1, 1)
        cnt = jnp.sum((u >= mid).astype(jnp.float32), axis=1, keepdims=True)
        ok = cnt >= kkf
        return jnp.where(ok, mid, lo), jnp.where(ok, hi, mid - 1)

    lo0 = jnp.zeros((_QBLK, 1), jnp.int32)
    hi0 = jnp.full((_QBLK, 1), 0x7F800000, jnp.int32)
    thr, _ = lax.fori_loop(0, 31, step, (lo0, hi0))

    gt = u > thr
    eq = jnp.where(u == thr, 1.0, 0.0)
    gtc = jnp.sum(gt.astype(jnp.float32), axis=1, keepdims=True)
    need = kkf - gtc
    inc = eq
    for d in (1, 2, 4, 8, 16, 32, 64, 128):
        inc = inc + jnp.concatenate(
            [jnp.zeros((_QBLK, d), jnp.float32), inc[:, :-d]], axis=1)
    prefix_excl = inc - eq
    sel = (vm >= 0.0) & (gt | ((eq > 0.5) & (prefix_excl < need)))
    vals = jnp.where(sel, 0.0, _NEG_INF).astype(jnp.float32)

    shifted = jnp.concatenate(
        [vals[:, _QBLK:], jnp.full((_QBLK, _QBLK), _NEG_INF, jnp.float32)],
        axis=1)
    band_ref[...] = jnp.where(i == 0, shifted, vals)


def _sc_body(band_hbm, out_hbm, band_v, tmpl, lsem, tsem, bsem):
    wid = lax.axis_index("s") * 2 + lax.axis_index("c")
    gbase = wid * _QBLK
    i_blk = lax.rem(wid, 16)
    coff = _QBLK * lax.max(i_blk - 1, 0)

    load = pltpu.async_copy(band_hbm.at[pl.ds(gbase, _QBLK)], band_v, lsem)

    neg = jnp.full((16,), _NEG_INF, jnp.float32)

    def fill(j, _):
        for rb in range(8):
            tmpl[rb, pl.ds(j * 16, 16)] = neg
        return _

    lax.fori_loop(0, 128, fill, None)

    blankets = [
        pltpu.async_copy(tmpl, out_hbm.at[pl.ds(gbase + 8 * t, 8)], tsem)
        for t in range(16)
    ]
    load.wait()
    for cp in blankets:
        cp.wait()
    pltpu.async_copy(
        band_v, out_hbm.at[pl.ds(gbase, _QBLK), pl.ds(coff, 2 * _QBLK)],
        bsem).wait()


def kernel(x, Wq, Wk):
    B, S, D = x.shape
    idim = Wq.shape[0]
    nblk = S // _QBLK
    band = pl.pallas_call(
        _band_body,
        grid=(B, nblk),
        in_specs=[
            pl.BlockSpec((1, _QBLK, D), lambda b, i: (b, i, 0)),
            pl.BlockSpec((1, _QBLK, D), lambda b, i: (b, lax.max(i - 1, 0), 0)),
            pl.BlockSpec((idim, D), lambda b, i: (0, 0)),
            pl.BlockSpec((idim, D), lambda b, i: (0, 0)),
        ],
        out_specs=pl.BlockSpec((_QBLK, 2 * _QBLK), lambda b, i: (b * nblk + i, 0)),
        out_shape=jax.ShapeDtypeStruct((B * S, 2 * _QBLK), jnp.float32),
        compiler_params=pltpu.CompilerParams(
            dimension_semantics=("parallel", "arbitrary"),
        ),
    )(x, x, Wq, Wk)

    sc_construct = pl.kernel(
        _sc_body,
        out_type=jax.ShapeDtypeStruct((B * S, S), jnp.float32),
        mesh=plsc.VectorSubcoreMesh(core_axis_name="c", subcore_axis_name="s",
                                    num_cores=2),
        scratch_types=[
            pltpu.VMEM((_QBLK, 2 * _QBLK), jnp.float32),
            pltpu.VMEM((8, S), jnp.float32),
            pltpu.SemaphoreType.DMA,
            pltpu.SemaphoreType.DMA,
            pltpu.SemaphoreType.DMA,
        ],
    )
    mask2d = sc_construct(band)
    return mask2d.reshape(B, 1, S, S)


# speedup vs baseline: 82.0630x; 1.3730x over previous
import jax
import jax.numpy as jnp
from jax import lax
from jax.experimental import pallas as pl
from jax.experimental.pallas import tpu as pltpu
from jax.experimental.pallas import tpu_sc as plsc

_WINDOW = 128
_QBLK = 128
_TILE = 512
_NEG_INF = float("-inf")

_DN = (((1,), (1,)), ((), ()))


def _band_body(xq_ref, xlo_ref, wq_ref, wk_ref, band_ref):
    i = pl.program_id(1)
    nsub = _TILE // _QBLK
    ncol = _TILE + _QBLK
    xq = xq_ref[0]
    xlo = xlo_ref[0]
    wq = wq_ref[...]
    wk = wk_ref[...]

    q = lax.dot_general(xq, wq, _DN, preferred_element_type=jnp.float32)
    k_mid = lax.dot_general(xq, wk, _DN, preferred_element_type=jnp.float32)
    k_lo = lax.dot_general(xlo, wk, _DN, preferred_element_type=jnp.float32)
    s_lo = lax.dot_general(q, k_lo, _DN, preferred_element_type=jnp.float32)
    s_mid = lax.dot_general(q, k_mid, _DN, preferred_element_type=jnp.float32)
    v = jnp.maximum(jnp.concatenate([s_lo, s_mid], axis=1), 0.0)

    r = lax.broadcasted_iota(jnp.int32, (_TILE, ncol), 0)
    c = lax.broadcasted_iota(jnp.int32, (_TILE, ncol), 1)
    valid = (c >= r + 1) & (c <= r + _WINDOW) & (_QBLK * (nsub * i - 1) + c >= 0)
    vm = jnp.where(valid, v, -1.0)

    rr = lax.broadcasted_iota(jnp.int32, (_TILE, 1), 0)
    length = jnp.minimum(_TILE * i + rr + 1, _WINDOW)
    kk = jnp.maximum(1, length // 2)
    kkf = kk.astype(jnp.float32)

    u = lax.bitcast_convert_type(vm + 0.0, jnp.int32)

    def step(_, lohi):
        lo, hi = lohi
        mid = lo + lax.shift_right_logical(hi - lo + 1, 1)
        cnt = jnp.sum((u >= mid).astype(jnp.float32), axis=1, keepdims=True)
        ok = cnt >= kkf
        return jnp.where(ok, mid, lo), jnp.where(ok, hi, mid - 1)

    lo0 = jnp.zeros((_TILE, 1), jnp.int32)
    hi0 = jnp.full((_TILE, 1), 0x7F800000, jnp.int32)
    thr, _ = lax.fori_loop(0, 31, step, (lo0, hi0))

    gt = u > thr
    eq = jnp.where(u == thr, 1.0, 0.0)
    gtc = jnp.sum(gt.astype(jnp.float32), axis=1, keepdims=True)
    need = kkf - gtc
    inc = eq
    d = 1
    while d < ncol:
        inc = inc + jnp.concatenate(
            [jnp.zeros((_TILE, d), jnp.float32), inc[:, :-d]], axis=1)
        d *= 2
    prefix_excl = inc - eq
    sel = (vm >= 0.0) & (gt | ((eq > 0.5) & (prefix_excl < need)))
    vals = jnp.where(sel, 0.0, _NEG_INF).astype(jnp.float32)

    for j in range(nsub):
        chunk = vals[_QBLK * j:_QBLK * (j + 1), _QBLK * j:_QBLK * j + 2 * _QBLK]
        if j == 0:
            shifted = jnp.concatenate(
                [chunk[:, _QBLK:],
                 jnp.full((_QBLK, _QBLK), _NEG_INF, jnp.float32)], axis=1)
            chunk = jnp.where(i == 0, shifted, chunk)
        band_ref[_QBLK * j:_QBLK * (j + 1), :] = chunk


def _sc_body(band_hbm, out_hbm, band_v, tmpl, lsem, tsem, bsem):
    wid = lax.axis_index("s") * 2 + lax.axis_index("c")
    gbase = wid * _QBLK
    i_blk = lax.rem(wid, 16)
    coff = _QBLK * lax.max(i_blk - 1, 0)

    load = pltpu.async_copy(band_hbm.at[pl.ds(gbase, _QBLK)], band_v, lsem)

    neg = jnp.full((16,), _NEG_INF, jnp.float32)

    def fill(j, carry):
        for rb in range(8):
            tmpl[rb, pl.ds(j * 16, 16)] = neg
        return carry

    lax.fori_loop(0, 128, fill, None)

    blankets = [
        pltpu.async_copy(tmpl, out_hbm.at[pl.ds(gbase + 8 * t, 8)], tsem)
        for t in range(16)
    ]
    load.wait()
    for cp in blankets:
        cp.wait()
    pltpu.async_copy(
        band_v, out_hbm.at[pl.ds(gbase, _QBLK), pl.ds(coff, 2 * _QBLK)],
        bsem).wait()


def kernel(x, Wq, Wk):
    B, S, D = x.shape
    idim = Wq.shape[0]
    ntile = S // _TILE
    band = pl.pallas_call(
        _band_body,
        grid=(B, ntile),
        in_specs=[
            pl.BlockSpec((1, _TILE, D), lambda b, i: (b, i, 0)),
            pl.BlockSpec((1, _QBLK, D),
                         lambda b, i: (b, lax.max(4 * i - 1, 0), 0)),
            pl.BlockSpec((idim, D), lambda b, i: (0, 0)),
            pl.BlockSpec((idim, D), lambda b, i: (0, 0)),
        ],
        out_specs=pl.BlockSpec((_TILE, 2 * _QBLK),
                               lambda b, i: (b * ntile + i, 0)),
        out_shape=jax.ShapeDtypeStruct((B * S, 2 * _QBLK), jnp.float32),
        compiler_params=pltpu.CompilerParams(
            dimension_semantics=("parallel", "arbitrary"),
        ),
    )(x, x, Wq, Wk)

    sc_construct = pl.kernel(
        _sc_body,
        out_type=jax.ShapeDtypeStruct((B * S, S), jnp.float32),
        mesh=plsc.VectorSubcoreMesh(core_axis_name="c", subcore_axis_name="s",
                                    num_cores=2),
        scratch_types=[
            pltpu.VMEM((_QBLK, 2 * _QBLK), jnp.float32),
            pltpu.VMEM((8, S), jnp.float32),
            pltpu.SemaphoreType.DMA,
            pltpu.SemaphoreType.DMA,
            pltpu.SemaphoreType.DMA,
        ],
    )
    mask2d = sc_construct(band)
    return mask2d.reshape(B, 1, S, S)


# speedup vs baseline: 95.6891x; 1.1660x over previous
import jax
import jax.numpy as jnp
from jax import lax
from jax.experimental import pallas as pl
from jax.experimental.pallas import tpu as pltpu
from jax.experimental.pallas import tpu_sc as plsc

_WINDOW = 128
_QBLK = 128
_TILE = 1024
_NEG_INF = float("-inf")

_DN = (((1,), (1,)), ((), ()))


def _band_body(xq_ref, xlo_ref, wq_ref, wk_ref, band_ref):
    i = pl.program_id(1)
    nsub = _TILE // _QBLK
    ncol = 2 * _QBLK
    xq = xq_ref[0]
    xlo = xlo_ref[0]
    wq = wq_ref[...]
    wk = wk_ref[...]

    q = lax.dot_general(xq, wq, _DN, preferred_element_type=jnp.float32)
    k_mid = lax.dot_general(xq, wk, _DN, preferred_element_type=jnp.float32)
    k_lo = lax.dot_general(xlo, wk, _DN, preferred_element_type=jnp.float32)

    groups = []
    for g in range(nsub):
        qg = q[_QBLK * g:_QBLK * (g + 1)]
        kprev = k_lo if g == 0 else k_mid[_QBLK * (g - 1):_QBLK * g]
        khigh = k_mid[_QBLK * g:_QBLK * (g + 1)]
        s_pre = lax.dot_general(qg, kprev, _DN,
                                preferred_element_type=jnp.float32)
        s_hi = lax.dot_general(qg, khigh, _DN,
                               preferred_element_type=jnp.float32)
        groups.append(jnp.concatenate([s_pre, s_hi], axis=1))
    v = jnp.maximum(jnp.concatenate(groups, axis=0), 0.0)

    r = lax.broadcasted_iota(jnp.int32, (_TILE, ncol), 0)
    rg = lax.rem(r, _QBLK)
    c = lax.broadcasted_iota(jnp.int32, (_TILE, ncol), 1)
    blk = nsub * i + r // _QBLK
    valid = (c >= rg + 1) & (c <= rg + _WINDOW) & (_QBLK * (blk - 1) + c >= 0)
    vm = jnp.where(valid, v, -1.0)

    length = jnp.minimum(_QBLK * blk[:, :1] + rg[:, :1] + 1, _WINDOW)
    kk = jnp.maximum(1, length // 2)
    kkf = kk.astype(jnp.float32)

    u = lax.bitcast_convert_type(vm + 0.0, jnp.int32)

    def step(_, lohi):
        lo, hi = lohi
        mid = lo + lax.shift_right_logical(hi - lo + 1, 1)
        cnt = jnp.sum((u >= mid).astype(jnp.float32), axis=1, keepdims=True)
        ok = cnt >= kkf
        return jnp.where(ok, mid, lo), jnp.where(ok, hi, mid - 1)

    lo0 = jnp.zeros((_TILE, 1), jnp.int32)
    hi0 = jnp.full((_TILE, 1), 0x7F800000, jnp.int32)
    thr, _ = lax.fori_loop(0, 31, step, (lo0, hi0))

    gt = u > thr
    eq = jnp.where(u == thr, 1.0, 0.0)
    gtc = jnp.sum(gt.astype(jnp.float32), axis=1, keepdims=True)
    need = kkf - gtc
    inc = eq
    d = 1
    while d < ncol:
        inc = inc + jnp.concatenate(
            [jnp.zeros((_TILE, d), jnp.float32), inc[:, :-d]], axis=1)
        d *= 2
    prefix_excl = inc - eq
    sel = (vm >= 0.0) & (gt | ((eq > 0.5) & (prefix_excl < need)))
    vals = jnp.where(sel, 0.0, _NEG_INF).astype(jnp.float32)

    first = vals[:_QBLK]
    shifted = jnp.concatenate(
        [first[:, _QBLK:],
         jnp.full((_QBLK, _QBLK), _NEG_INF, jnp.float32)], axis=1)
    band_ref[:_QBLK, :] = jnp.where(i == 0, shifted, first)
    band_ref[_QBLK:, :] = vals[_QBLK:]


def _sc_body(band_hbm, out_hbm, band_v, tmpl, lsem, tsem, bsem):
    wid = lax.axis_index("s") * 2 + lax.axis_index("c")
    gbase = wid * _QBLK
    i_blk = lax.rem(wid, 16)
    coff = _QBLK * lax.max(i_blk - 1, 0)

    load = pltpu.async_copy(band_hbm.at[pl.ds(gbase, _QBLK)], band_v, lsem)

    neg = jnp.full((16,), _NEG_INF, jnp.float32)

    def fill(j, carry):
        for rb in range(8):
            tmpl[rb, pl.ds(j * 16, 16)] = neg
        return carry

    lax.fori_loop(0, 128, fill, None)

    blankets = [
        pltpu.async_copy(tmpl, out_hbm.at[pl.ds(gbase + 8 * t, 8)], tsem)
        for t in range(16)
    ]
    load.wait()
    for cp in blankets:
        cp.wait()
    pltpu.async_copy(
        band_v, out_hbm.at[pl.ds(gbase, _QBLK), pl.ds(coff, 2 * _QBLK)],
        bsem).wait()


def kernel(x, Wq, Wk):
    B, S, D = x.shape
    idim = Wq.shape[0]
    ntile = S // _TILE
    band = pl.pallas_call(
        _band_body,
        grid=(B, ntile),
        in_specs=[
            pl.BlockSpec((1, _TILE, D), lambda b, i: (b, i, 0)),
            pl.BlockSpec((1, _QBLK, D),
                         lambda b, i: (b, lax.max(8 * i - 1, 0), 0)),
            pl.BlockSpec((idim, D), lambda b, i: (0, 0)),
            pl.BlockSpec((idim, D), lambda b, i: (0, 0)),
        ],
        out_specs=pl.BlockSpec((_TILE, 2 * _QBLK),
                               lambda b, i: (b * ntile + i, 0)),
        out_shape=jax.ShapeDtypeStruct((B * S, 2 * _QBLK), jnp.float32),
        compiler_params=pltpu.CompilerParams(
            dimension_semantics=("parallel", "arbitrary"),
        ),
    )(x, x, Wq, Wk)

    sc_construct = pl.kernel(
        _sc_body,
        out_type=jax.ShapeDtypeStruct((B * S, S), jnp.float32),
        mesh=plsc.VectorSubcoreMesh(core_axis_name="c", subcore_axis_name="s",
                                    num_cores=2),
        scratch_types=[
            pltpu.VMEM((_QBLK, 2 * _QBLK), jnp.float32),
            pltpu.VMEM((8, S), jnp.float32),
            pltpu.SemaphoreType.DMA,
            pltpu.SemaphoreType.DMA,
            pltpu.SemaphoreType.DMA,
        ],
    )
    mask2d = sc_construct(band)
    return mask2d.reshape(B, 1, S, S)
